# Initial kernel scaffold; baseline (speedup 1.0000x reference)
import jax
import jax.numpy as jnp
from jax.experimental import pallas as pl


def kernel(a):
    raise NotImplementedError("write your pallas kernel here")


# speedup vs baseline: 32.4126x; 32.4126x over previous
import numpy as np
import jax
import jax.numpy as jnp
from jax.experimental import pallas as pl


_U32 = np.uint32


def _np_rotl(x, r):
    return ((x << _U32(r)) | (x >> _U32(32 - r))).astype(np.uint32)


def _np_threefry(k0, k1, x0, x1):
    with np.errstate(over="ignore"):
        k0 = _U32(k0)
        k1 = _U32(k1)
        ks2 = _U32(np.uint32(k0) ^ np.uint32(k1) ^ _U32(0x1BD11BDA))
        ks = [k0, k1, ks2]
        rot = [[13, 15, 26, 6], [17, 29, 16, 24]]
        x0 = x0.astype(np.uint32) + k0
        x1 = x1.astype(np.uint32) + k1
        for i in range(5):
            for r in rot[i % 2]:
                x0 = x0 + x1
                x1 = _np_rotl(x1, r)
                x1 = x1 ^ x0
            x0 = x0 + ks[(i + 1) % 3]
            x1 = x1 + ks[(i + 2) % 3] + _U32(i + 1)
        return x0, x1


def _np_split(keypair, num):
    cnt = np.arange(num, dtype=np.uint32)
    o0, o1 = _np_threefry(keypair[0], keypair[1], np.zeros(num, np.uint32), cnt)
    return np.stack([o0, o1], axis=1)


_BASE = np.array([0, 1], dtype=np.uint32)
_KS = _np_split(_BASE, 19)
_K6 = _np_split(_KS[6], 2)
_K17B = _np_split(_KS[17], 2)[1]
_K7B = _np_split(_KS[7], 2)[1]

_N_POIS = 24
_SUB = np.zeros((_N_POIS, 2), np.uint32)
_rng = _KS[3]
for _i in range(_N_POIS):
    _pr = _np_split(_rng, 2)
    _rng, _SUB[_i] = _pr[0], _pr[1]


_ROWS = 4
_K0P = np.zeros((_ROWS, 128), np.uint32)
_K1P = np.zeros((_ROWS, 128), np.uint32)
_CNTP = np.zeros((_ROWS, 128), np.uint32)

_LAYOUT = {}


def _place(name, key, n, row, col):
    _K0P[row, col:col + n] = key[0]
    _K1P[row, col:col + n] = key[1]
    _CNTP[row, col:col + n] = np.arange(n, dtype=np.uint32)
    _LAYOUT[name] = (row, col, n)


_r0 = [("bern", _KS[0]), ("rndlk", _KS[5]), ("ril", _K7B), ("rnl", _KS[9]),
       ("abern", _KS[11]), ("cauchy", _KS[12]), ("expo", _KS[13]),
       ("geom", _KS[14]), ("lognm", _KS[15]), ("norm", _KS[16]),
       ("rnd", _K17B), ("unif", _KS[18])]
for _j, (_nm, _k) in enumerate(_r0):
    _place(_nm, _k, 9, 0, 9 * _j)
_r1 = [("gumb", _KS[1], 4), ("nrm4", _KS[2], 4), ("riA", _K6[0], 4),
       ("riB", _K6[1], 4), ("rand4", _KS[8], 4), ("r23", _KS[4], 6)]
_c = 0
for _nm, _k, _n in _r1:
    _place(_nm, _k, _n, 1, _c)
    _c += _n
for _i in range(_N_POIS):
    _row, _col = (2, 9 * _i) if _i < 14 else (3, 9 * (_i - 14))
    _place(f"pois{_i}", _SUB[_i], 9, _row, _col)

_K0J = jnp.asarray(_K0P)
_K1J = jnp.asarray(_K1P)
_CNTJ = jnp.asarray(_CNTP)

_LO_NRM = np.float32(np.nextafter(np.float32(-1.0), np.float32(0.0)))
_SPAN_NRM = np.float32(np.float32(1.0) - _LO_NRM)
_SQRT2 = np.float32(np.sqrt(2.0))
_LOGITS = np.where(np.array([0., 10., 3., 0.]) > 0,
                   np.log(np.maximum(np.array([0., 10., 3., 0.]), 1e-30)),
                   -1e30).astype(np.float32)


def _erfinv(x):
    w = -jnp.log((1.0 - x) * (1.0 + x))
    wc = w - 2.5
    p = jnp.float32(2.81022636e-08)
    for c in (3.43273939e-07, -3.5233877e-06, -4.39150654e-06, 0.00021858087,
              -0.00125372503, -0.00417768164, 0.246640727, 1.50140941):
        p = jnp.float32(c) + p * wc
    wt = jnp.sqrt(jnp.maximum(w, 5.0)) - 3.0
    q = jnp.float32(-0.000200214257)
    for c in (0.000100950558, 0.00134934322, -0.00367342844, 0.00573950773,
              -0.0076224613, 0.00943887047, 1.00167406, 2.83297682):
        q = jnp.float32(c) + q * wt
    return jnp.where(w < 5.0, p, q) * x


def _tanpoly(t):
    t2 = t * t
    s = t * (1.0 + t2 * (-1.0 / 6 + t2 * (1.0 / 120 + t2 * (-1.0 / 5040
             + t2 * (1.0 / 362880)))))
    c = 1.0 + t2 * (-0.5 + t2 * (1.0 / 24 + t2 * (-1.0 / 720
             + t2 * (1.0 / 40320 + t2 * (-1.0 / 3628800)))))
    return s / c


def _body(k0_ref, k1_ref, cnt_ref, a_ref, o_ref):
    k0 = k0_ref[...]
    k1 = k1_ref[...]
    ks2 = k0 ^ k1 ^ jnp.uint32(0x1BD11BDA)
    ks = [k0, k1, ks2]
    rot = [[13, 15, 26, 6], [17, 29, 16, 24]]
    x0 = k0
    x1 = cnt_ref[...] + k1
    for i in range(5):
        for r in rot[i % 2]:
            x0 = x0 + x1
            x1 = (x1 << jnp.uint32(r)) | (x1 >> jnp.uint32(32 - r))
            x1 = x1 ^ x0
        x0 = x0 + ks[(i + 1) % 3]
        x1 = x1 + ks[(i + 2) % 3] + jnp.uint32(i + 1)
    bits = x0 ^ x1
    fb = (bits >> jnp.uint32(9)) | jnp.uint32(0x3F800000)
    u = jax.lax.bitcast_convert_type(fb, jnp.float32) - 1.0

    def useg(name):
        r, c, n = _LAYOUT[name]
        return u[r:r + 1, c:c + n]

    def bseg(name):
        r, c, n = _LAYOUT[name]
        return bits[r:r + 1, c:c + n]

    def clip(x):
        return jnp.clip(x, 1e-12, 1.0 - 1e-12)

    def nrm(name):
        x = jnp.maximum(_LO_NRM, useg(name) * _SPAN_NRM + _LO_NRM)
        return _SQRT2 * _erfinv(x)

    af = a_ref[0:1, 0:9]
    m = [None] * 19
    m[0] = jnp.sum((useg("bern") < af).astype(jnp.float32)) / 9.0
    m[11] = jnp.sum((useg("abern") < af).astype(jnp.float32)) / 9.0
    g = -jnp.log(-jnp.log(clip(useg("gumb"))))
    i4 = jax.lax.broadcasted_iota(jnp.int32, (1, 4), 1)
    logits = jnp.where(i4 == 1, jnp.float32(_LOGITS[1]),
                       jnp.where(i4 == 2, jnp.float32(_LOGITS[2]),
                                 jnp.float32(-1e30)))
    t = logits + g
    idx = i4.astype(jnp.float32)
    m1 = jnp.max(t)
    i1 = jnp.sum(jnp.where(t == m1, idx, 0.0))
    t2 = jnp.where(t == m1, jnp.float32(-3e38), t)
    i2 = jnp.sum(jnp.where(t2 == jnp.max(t2), idx, 0.0))
    m[1] = (i1 + i2) * 0.5
    m[2] = jnp.sum(2.0 + 3.0 * nrm("nrm4")) / 4.0
    lp = jnp.zeros((1, 9), jnp.float32)
    cnt = jnp.zeros((1, 9), jnp.float32)
    for i in range(_N_POIS):
        cnt = cnt + (lp > -af).astype(jnp.float32)
        lp = lp + jnp.log(useg(f"pois{i}"))
    m[3] = (jnp.sum(cnt) - 9.0) / 9.0
    m[4] = jnp.sum(useg("r23")) / 6.0
    m[5] = jnp.sum(useg("rndlk")) / 9.0
    m[8] = jnp.sum(useg("rand4")) / 4.0
    m[18] = jnp.sum(useg("unif")) / 9.0
    hi = bseg("riA")
    lo = bseg("riB")
    off = ((hi % jnp.uint32(10)) * jnp.uint32(6) + (lo % jnp.uint32(10))) \
        % jnp.uint32(10)
    m[6] = jnp.sum(off.astype(jnp.float32)) / 4.0
    m[7] = jnp.sum((bseg("ril") & jnp.uint32(3)).astype(jnp.float32)) / 9.0
    m[17] = jnp.sum((bseg("rnd") & jnp.uint32(0xFFFFFF))
                    .astype(jnp.float32)) / 9.0
    m[9] = jnp.sum(nrm("rnl")) / 9.0
    m[16] = jnp.sum(nrm("norm")) / 9.0
    m[10] = jnp.float32(1.5)
    m[12] = jnp.sum(_tanpoly(jnp.float32(np.pi) * (clip(useg("cauchy")) - 0.5))) / 9.0
    m[13] = jnp.sum(-jnp.log(1.0 - useg("expo"))) / 9.0
    m[14] = jnp.sum(jnp.floor(jnp.log(clip(useg("geom")))
                              / jnp.float32(np.log(0.5))) + 1.0) / 9.0
    m[15] = jnp.sum(jnp.exp(1.0 + 2.0 * nrm("lognm"))) / 9.0

    cols = jax.lax.broadcasted_iota(jnp.int32, (1, 128), 1)
    acc = jnp.zeros((1, 128), jnp.float32)
    for s in range(19):
        acc = acc + jnp.where(cols == s, m[s], 0.0)
    o_ref[...] = acc


def kernel(a):
    a_row = jnp.zeros((1, 128), jnp.float32).at[0, :9].set(a.reshape(-1))
    out = pl.pallas_call(
        _body,
        out_shape=jax.ShapeDtypeStruct((1, 128), jnp.float32),
    )(_K0J, _K1J, _CNTJ, a_row)
    return out[0, :19]
